# baseline (speedup 1.0000x reference)
import functools

import jax
import jax.numpy as jnp
from jax import lax
from jax.experimental import pallas as pl
from jax.experimental.pallas import tpu as pltpu
from jax.experimental.pallas import tpu_sc as plsc

_N_NODES = 50000
_N_EDGES = 800000
_NG = 50
_HID = 128
_EMB = 96
_TAGW = 32

_CTAB = 256


def _fuse_body(emb_ref, tagt_ref, w_ref, b_ref, c_ref):
    a = jnp.dot(emb_ref[:], w_ref[:_EMB, :], preferred_element_type=jnp.float32)
    bt = jnp.dot(tagt_ref[:], w_ref[_EMB:, :], preferred_element_type=jnp.float32)
    k = lax.broadcasted_iota(jnp.int32, (_CTAB, 1), 0)
    zi = k // 3
    ti = k - zi * 3
    oz = (lax.broadcasted_iota(jnp.int32, (_CTAB, 85), 1) == zi).astype(jnp.float32)
    ot = (lax.broadcasted_iota(jnp.int32, (_CTAB, 3), 1) == ti).astype(jnp.float32)
    c_ref[:] = (
        jnp.dot(oz, a, preferred_element_type=jnp.float32)
        + jnp.dot(ot, bt, preferred_element_type=jnp.float32)
        + b_ref[:]
    )


def _build_fused_table(emb_table, tag_table, lin_W, lin_b, interpret=False):
    return pl.pallas_call(
        _fuse_body,
        out_shape=jax.ShapeDtypeStruct((_CTAB, _HID), jnp.float32),
        interpret=interpret,
    )(emb_table, tag_table, lin_W, lin_b.reshape(1, _HID))


_BE = 6400


def _edge_body(rel_ref, attr_ref, wr_ref, wa_ref, b_ref, out_ref):
    attr_t = attr_ref[:].astype(jnp.bfloat16)
    rel_t = rel_ref[:].astype(jnp.bfloat16)
    wa = wa_ref[:].astype(jnp.bfloat16)
    wr = wr_ref[:].astype(jnp.bfloat16)
    dn = (((0,), (0,)), ((), ()))
    out_ref[:] = (
        lax.dot_general(attr_t, wa, dn, preferred_element_type=jnp.float32)
        + lax.dot_general(rel_t, wr, dn, preferred_element_type=jnp.float32)
        + b_ref[:]
    )


def _edge_proj(rel_pos, edge_attr, lin_e_W, lin_e_b, interpret=False):
    n = rel_pos.shape[0]
    grid = (n // _BE,)
    return pl.pallas_call(
        _edge_body,
        grid=grid,
        in_specs=[
            pl.BlockSpec((3, _BE), lambda i: (0, i)),
            pl.BlockSpec((_NG, _BE), lambda i: (0, i)),
            pl.BlockSpec((3, _HID), lambda i: (0, 0)),
            pl.BlockSpec((_NG, _HID), lambda i: (0, 0)),
            pl.BlockSpec((1, _HID), lambda i: (0, 0)),
        ],
        out_specs=pl.BlockSpec((_BE, _HID), lambda i: (i, 0)),
        out_shape=jax.ShapeDtypeStruct((n, _HID), jnp.float32),
        interpret=interpret,
    )(rel_pos.T, edge_attr.T, lin_e_W[:3], lin_e_W[3:], lin_e_b.reshape(1, _HID))


_UNIT = 128
_UNITS_PER_W = 13
_FULL_W = 30
_ROWS_PER_W = _UNITS_PER_W * _UNIT
_TAIL_BASE = _FULL_W * _ROWS_PER_W
_TAIL = _N_NODES - _TAIL_BASE
_RING = 4
_INFLIGHT = 3


def _sc_gather_body(z_hbm, tag_hbm, c_hbm, out_hbm, z_v, tag_v, idx2_v, rows_v,
                    gs0, gs1, gs2, gs3, ws0, ws1, ws2, ws3):
    c = lax.axis_index("c")
    s = lax.axis_index("s")
    w = s * 2 + c
    gsems = [gs0, gs1, gs2, gs3]
    wsems = [ws0, ws1, ws2, ws3]

    @pl.when(w < _FULL_W)
    def _():
        base0 = w * _ROWS_PER_W
        pltpu.sync_copy(z_hbm.at[pl.ds(base0, _ROWS_PER_W)], z_v)
        pltpu.sync_copy(tag_hbm.at[pl.ds(base0, _ROWS_PER_W)], tag_v)
        for u in range(_UNITS_PER_W):
            for j in range(_UNIT // 16):
                src = pl.ds(u * _UNIT + j * 16, 16)
                idx2_v[u, pl.ds(j * 16, 16)] = z_v[src] * 3 + tag_v[src]

        def gather(u):
            return pltpu.async_copy(
                c_hbm.at[idx2_v.at[u]], rows_v.at[u % _RING], gsems[u % _RING])

        def writeback(u):
            return pltpu.async_copy(
                rows_v.at[u % _RING],
                out_hbm.at[pl.ds(base0 + u * _UNIT, _UNIT)],
                wsems[u % _RING])

        gs = {}
        wbs = {}
        for p in range(_INFLIGHT):
            gs[p] = gather(p)
        for u in range(_UNITS_PER_W):
            gs[u].wait()
            wbs[u] = writeback(u)
            nxt = u + _INFLIGHT
            if nxt < _UNITS_PER_W:
                if nxt >= _RING:
                    wbs[nxt - _RING].wait()
                gs[nxt] = gather(nxt)
        for u in range(_UNITS_PER_W - _RING, _UNITS_PER_W):
            wbs[u].wait()

    @pl.when(w == _FULL_W)
    def _():
        pltpu.sync_copy(z_hbm.at[pl.ds(_TAIL_BASE, _TAIL)], z_v.at[pl.ds(0, _TAIL)])
        pltpu.sync_copy(tag_hbm.at[pl.ds(_TAIL_BASE, _TAIL)], tag_v.at[pl.ds(0, _TAIL)])
        for j in range(_TAIL // 16):
            sl = pl.ds(j * 16, 16)
            idx2_v[0, sl] = z_v[sl] * 3 + tag_v[sl]
        pltpu.async_copy(
            c_hbm.at[idx2_v.at[0, pl.ds(0, _TAIL)]],
            rows_v.at[0, pl.ds(0, _TAIL)], gs0).wait()
        pltpu.sync_copy(rows_v.at[0, pl.ds(0, _TAIL)],
                        out_hbm.at[pl.ds(_TAIL_BASE, _TAIL)])


def _node_gather(z, tag, c_table):
    mesh = plsc.VectorSubcoreMesh(core_axis_name="c", subcore_axis_name="s")
    return pl.kernel(
        _sc_gather_body,
        out_type=jax.ShapeDtypeStruct((_N_NODES, _HID), jnp.float32),
        mesh=mesh,
        scratch_types=[
            pltpu.VMEM((_ROWS_PER_W,), jnp.int32),
            pltpu.VMEM((_ROWS_PER_W,), jnp.int32),
            pltpu.VMEM((_UNITS_PER_W, _UNIT), jnp.int32),
            pltpu.VMEM((_RING, _UNIT, _HID), jnp.float32),
        ] + [pltpu.SemaphoreType.DMA] * 8,
    )(z, tag, c_table)


def kernel(z, rel_pos, edge_attr, tag, emb_table, tag_table, lin_W, lin_b, lin_e_W, lin_e_b):
    c_table = _build_fused_table(emb_table, tag_table, lin_W, lin_b)
    h = _node_gather(z.astype(jnp.int32), tag.astype(jnp.int32), c_table)
    e = _edge_proj(rel_pos, edge_attr, lin_e_W, lin_e_b)
    return (h, e)


# speedup vs baseline: 1.1413x; 1.1413x over previous
import functools

import jax
import jax.numpy as jnp
from jax import lax
from jax.experimental import pallas as pl
from jax.experimental.pallas import tpu as pltpu
from jax.experimental.pallas import tpu_sc as plsc

_N_NODES = 50000
_N_EDGES = 800000
_NG = 50
_HID = 128
_EMB = 96
_TAGW = 32

_CTAB = 256


def _fuse_body(emb_ref, tagt_ref, w_ref, b_ref, c_ref):
    a = jnp.dot(emb_ref[:], w_ref[:_EMB, :], preferred_element_type=jnp.float32)
    bt = jnp.dot(tagt_ref[:], w_ref[_EMB:, :], preferred_element_type=jnp.float32)
    k = lax.broadcasted_iota(jnp.int32, (_CTAB, 1), 0)
    zi = k // 3
    ti = k - zi * 3
    oz = (lax.broadcasted_iota(jnp.int32, (_CTAB, 85), 1) == zi).astype(jnp.float32)
    ot = (lax.broadcasted_iota(jnp.int32, (_CTAB, 3), 1) == ti).astype(jnp.float32)
    c_ref[:] = (
        jnp.dot(oz, a, preferred_element_type=jnp.float32)
        + jnp.dot(ot, bt, preferred_element_type=jnp.float32)
        + b_ref[:]
    )


def _build_fused_table(emb_table, tag_table, lin_W, lin_b, interpret=False):
    return pl.pallas_call(
        _fuse_body,
        out_shape=jax.ShapeDtypeStruct((_CTAB, _HID), jnp.float32),
        interpret=interpret,
    )(emb_table, tag_table, lin_W, lin_b.reshape(1, _HID))


_BE = 16000


def _edge_body(rel_ref, attr_ref, wr_ref, wa_ref, b_ref, out_ref):
    attr_t = attr_ref[:].astype(jnp.bfloat16)
    rel_t = rel_ref[:].astype(jnp.bfloat16)
    wa = wa_ref[:].astype(jnp.bfloat16)
    wr = wr_ref[:].astype(jnp.bfloat16)
    dn = (((0,), (0,)), ((), ()))
    out_ref[:] = (
        lax.dot_general(attr_t, wa, dn, preferred_element_type=jnp.float32)
        + lax.dot_general(rel_t, wr, dn, preferred_element_type=jnp.float32)
        + b_ref[:]
    )


def _edge_proj(rel_pos, edge_attr, lin_e_W, lin_e_b, interpret=False):
    n = rel_pos.shape[0]
    grid = (n // _BE,)
    return pl.pallas_call(
        _edge_body,
        grid=grid,
        in_specs=[
            pl.BlockSpec((3, _BE), lambda i: (0, i)),
            pl.BlockSpec((_NG, _BE), lambda i: (0, i)),
            pl.BlockSpec((3, _HID), lambda i: (0, 0)),
            pl.BlockSpec((_NG, _HID), lambda i: (0, 0)),
            pl.BlockSpec((1, _HID), lambda i: (0, 0)),
        ],
        out_specs=pl.BlockSpec((_BE, _HID), lambda i: (i, 0)),
        out_shape=jax.ShapeDtypeStruct((n, _HID), jnp.float32),
        interpret=interpret,
    )(rel_pos.T, edge_attr.T, lin_e_W[:3], lin_e_W[3:], lin_e_b.reshape(1, _HID))


_UNIT = 128
_UNITS_PER_W = 13
_FULL_W = 30
_ROWS_PER_W = _UNITS_PER_W * _UNIT
_TAIL_BASE = _FULL_W * _ROWS_PER_W
_TAIL = _N_NODES - _TAIL_BASE
_RING = 4
_INFLIGHT = 3


def _sc_gather_body(z_hbm, tag_hbm, c_hbm, out_hbm, z_v, tag_v, idx2_v, rows_v,
                    gs0, gs1, gs2, gs3, ws0, ws1, ws2, ws3):
    c = lax.axis_index("c")
    s = lax.axis_index("s")
    w = s * 2 + c
    gsems = [gs0, gs1, gs2, gs3]
    wsems = [ws0, ws1, ws2, ws3]

    @pl.when(w < _FULL_W)
    def _():
        base0 = w * _ROWS_PER_W
        pltpu.sync_copy(z_hbm.at[pl.ds(base0, _ROWS_PER_W)], z_v)
        pltpu.sync_copy(tag_hbm.at[pl.ds(base0, _ROWS_PER_W)], tag_v)
        for u in range(_UNITS_PER_W):
            for j in range(_UNIT // 16):
                src = pl.ds(u * _UNIT + j * 16, 16)
                idx2_v[u, pl.ds(j * 16, 16)] = z_v[src] * 3 + tag_v[src]

        def gather(u):
            return pltpu.async_copy(
                c_hbm.at[idx2_v.at[u]], rows_v.at[u % _RING], gsems[u % _RING])

        def writeback(u):
            return pltpu.async_copy(
                rows_v.at[u % _RING],
                out_hbm.at[pl.ds(base0 + u * _UNIT, _UNIT)],
                wsems[u % _RING])

        gs = {}
        wbs = {}
        for p in range(_INFLIGHT):
            gs[p] = gather(p)
        for u in range(_UNITS_PER_W):
            gs[u].wait()
            wbs[u] = writeback(u)
            nxt = u + _INFLIGHT
            if nxt < _UNITS_PER_W:
                if nxt >= _RING:
                    wbs[nxt - _RING].wait()
                gs[nxt] = gather(nxt)
        for u in range(_UNITS_PER_W - _RING, _UNITS_PER_W):
            wbs[u].wait()

    @pl.when(w == _FULL_W)
    def _():
        pltpu.sync_copy(z_hbm.at[pl.ds(_TAIL_BASE, _TAIL)], z_v.at[pl.ds(0, _TAIL)])
        pltpu.sync_copy(tag_hbm.at[pl.ds(_TAIL_BASE, _TAIL)], tag_v.at[pl.ds(0, _TAIL)])
        for j in range(_TAIL // 16):
            sl = pl.ds(j * 16, 16)
            idx2_v[0, sl] = z_v[sl] * 3 + tag_v[sl]
        pltpu.async_copy(
            c_hbm.at[idx2_v.at[0, pl.ds(0, _TAIL)]],
            rows_v.at[0, pl.ds(0, _TAIL)], gs0).wait()
        pltpu.sync_copy(rows_v.at[0, pl.ds(0, _TAIL)],
                        out_hbm.at[pl.ds(_TAIL_BASE, _TAIL)])


def _node_gather(z, tag, c_table):
    mesh = plsc.VectorSubcoreMesh(core_axis_name="c", subcore_axis_name="s")
    return pl.kernel(
        _sc_gather_body,
        out_type=jax.ShapeDtypeStruct((_N_NODES, _HID), jnp.float32),
        mesh=mesh,
        scratch_types=[
            pltpu.VMEM((_ROWS_PER_W,), jnp.int32),
            pltpu.VMEM((_ROWS_PER_W,), jnp.int32),
            pltpu.VMEM((_UNITS_PER_W, _UNIT), jnp.int32),
            pltpu.VMEM((_RING, _UNIT, _HID), jnp.float32),
        ] + [pltpu.SemaphoreType.DMA] * 8,
    )(z, tag, c_table)


def kernel(z, rel_pos, edge_attr, tag, emb_table, tag_table, lin_W, lin_b, lin_e_W, lin_e_b):
    c_table = _build_fused_table(emb_table, tag_table, lin_W, lin_b)
    h = _node_gather(z.astype(jnp.int32), tag.astype(jnp.int32), c_table)
    e = _edge_proj(rel_pos, edge_attr, lin_e_W, lin_e_b)
    return (h, e)


# speedup vs baseline: 1.1719x; 1.0268x over previous
import functools

import jax
import jax.numpy as jnp
from jax import lax
from jax.experimental import pallas as pl
from jax.experimental.pallas import tpu as pltpu
from jax.experimental.pallas import tpu_sc as plsc

_N_NODES = 50000
_N_EDGES = 800000
_NG = 50
_HID = 128
_EMB = 96
_TAGW = 32

_CTAB = 256


def _fuse_body(emb_ref, tagt_ref, w_ref, b_ref, c_ref):
    a = jnp.dot(emb_ref[:], w_ref[:_EMB, :], preferred_element_type=jnp.float32)
    bt = jnp.dot(tagt_ref[:], w_ref[_EMB:, :], preferred_element_type=jnp.float32)
    k = lax.broadcasted_iota(jnp.int32, (_CTAB, 1), 0)
    zi = k // 3
    ti = k - zi * 3
    oz = (lax.broadcasted_iota(jnp.int32, (_CTAB, 85), 1) == zi).astype(jnp.float32)
    ot = (lax.broadcasted_iota(jnp.int32, (_CTAB, 3), 1) == ti).astype(jnp.float32)
    c_ref[:] = (
        jnp.dot(oz, a, preferred_element_type=jnp.float32)
        + jnp.dot(ot, bt, preferred_element_type=jnp.float32)
        + b_ref[:]
    )


def _build_fused_table(emb_table, tag_table, lin_W, lin_b, interpret=False):
    return pl.pallas_call(
        _fuse_body,
        out_shape=jax.ShapeDtypeStruct((_CTAB, _HID), jnp.float32),
        interpret=interpret,
    )(emb_table, tag_table, lin_W, lin_b.reshape(1, _HID))


_BE = 32000


def _edge_body(rel_ref, attr_ref, w_ref, b_ref, out_ref):
    attr_t = attr_ref[:].astype(jnp.bfloat16)
    rel_t = rel_ref[:].astype(jnp.bfloat16)
    w = w_ref[:].astype(jnp.bfloat16)
    wr = w[0:3, :]
    wa = w[3:53, :]
    dn = (((0,), (0,)), ((), ()))
    out_ref[:] = (
        lax.dot_general(attr_t, wa, dn, preferred_element_type=jnp.float32)
        + lax.dot_general(rel_t, wr, dn, preferred_element_type=jnp.float32)
        + b_ref[:]
    )


def _edge_proj(rel_pos, edge_attr, lin_e_W, lin_e_b, interpret=False):
    n = rel_pos.shape[0]
    grid = (n // _BE,)
    return pl.pallas_call(
        _edge_body,
        grid=grid,
        in_specs=[
            pl.BlockSpec((3, _BE), lambda i: (0, i)),
            pl.BlockSpec((_NG, _BE), lambda i: (0, i)),
            pl.BlockSpec((_NG + 3, _HID), lambda i: (0, 0)),
            pl.BlockSpec((1, _HID), lambda i: (0, 0)),
        ],
        out_specs=pl.BlockSpec((_BE, _HID), lambda i: (i, 0)),
        out_shape=jax.ShapeDtypeStruct((n, _HID), jnp.float32),
        interpret=interpret,
    )(rel_pos.T, edge_attr.T, lin_e_W, lin_e_b.reshape(1, _HID))


_UNIT = 128
_UNITS_PER_W = 13
_FULL_W = 30
_ROWS_PER_W = _UNITS_PER_W * _UNIT
_TAIL_BASE = _FULL_W * _ROWS_PER_W
_TAIL = _N_NODES - _TAIL_BASE
_RING = 4
_INFLIGHT = 3


def _sc_gather_body(z_hbm, tag_hbm, c_hbm, out_hbm, z_v, tag_v, idx2_v, rows_v,
                    gs0, gs1, gs2, gs3, ws0, ws1, ws2, ws3):
    c = lax.axis_index("c")
    s = lax.axis_index("s")
    w = s * 2 + c
    gsems = [gs0, gs1, gs2, gs3]
    wsems = [ws0, ws1, ws2, ws3]

    @pl.when(w < _FULL_W)
    def _():
        base0 = w * _ROWS_PER_W
        pltpu.sync_copy(z_hbm.at[pl.ds(base0, _ROWS_PER_W)], z_v)
        pltpu.sync_copy(tag_hbm.at[pl.ds(base0, _ROWS_PER_W)], tag_v)
        for u in range(_UNITS_PER_W):
            for j in range(_UNIT // 16):
                src = pl.ds(u * _UNIT + j * 16, 16)
                idx2_v[u, pl.ds(j * 16, 16)] = z_v[src] * 3 + tag_v[src]

        def gather(u):
            return pltpu.async_copy(
                c_hbm.at[idx2_v.at[u]], rows_v.at[u % _RING], gsems[u % _RING])

        def writeback(u):
            return pltpu.async_copy(
                rows_v.at[u % _RING],
                out_hbm.at[pl.ds(base0 + u * _UNIT, _UNIT)],
                wsems[u % _RING])

        gs = {}
        wbs = {}
        for p in range(_INFLIGHT):
            gs[p] = gather(p)
        for u in range(_UNITS_PER_W):
            gs[u].wait()
            wbs[u] = writeback(u)
            nxt = u + _INFLIGHT
            if nxt < _UNITS_PER_W:
                if nxt >= _RING:
                    wbs[nxt - _RING].wait()
                gs[nxt] = gather(nxt)
        for u in range(_UNITS_PER_W - _RING, _UNITS_PER_W):
            wbs[u].wait()

    @pl.when(w == _FULL_W)
    def _():
        pltpu.sync_copy(z_hbm.at[pl.ds(_TAIL_BASE, _TAIL)], z_v.at[pl.ds(0, _TAIL)])
        pltpu.sync_copy(tag_hbm.at[pl.ds(_TAIL_BASE, _TAIL)], tag_v.at[pl.ds(0, _TAIL)])
        for j in range(_TAIL // 16):
            sl = pl.ds(j * 16, 16)
            idx2_v[0, sl] = z_v[sl] * 3 + tag_v[sl]
        pltpu.async_copy(
            c_hbm.at[idx2_v.at[0, pl.ds(0, _TAIL)]],
            rows_v.at[0, pl.ds(0, _TAIL)], gs0).wait()
        pltpu.sync_copy(rows_v.at[0, pl.ds(0, _TAIL)],
                        out_hbm.at[pl.ds(_TAIL_BASE, _TAIL)])


def _node_gather(z, tag, c_table):
    mesh = plsc.VectorSubcoreMesh(core_axis_name="c", subcore_axis_name="s")
    return pl.kernel(
        _sc_gather_body,
        out_type=jax.ShapeDtypeStruct((_N_NODES, _HID), jnp.float32),
        mesh=mesh,
        scratch_types=[
            pltpu.VMEM((_ROWS_PER_W,), jnp.int32),
            pltpu.VMEM((_ROWS_PER_W,), jnp.int32),
            pltpu.VMEM((_UNITS_PER_W, _UNIT), jnp.int32),
            pltpu.VMEM((_RING, _UNIT, _HID), jnp.float32),
        ] + [pltpu.SemaphoreType.DMA] * 8,
    )(z, tag, c_table)


def kernel(z, rel_pos, edge_attr, tag, emb_table, tag_table, lin_W, lin_b, lin_e_W, lin_e_b):
    c_table = _build_fused_table(emb_table, tag_table, lin_W, lin_b)
    h = _node_gather(z.astype(jnp.int32), tag.astype(jnp.int32), c_table)
    e = _edge_proj(rel_pos, edge_attr, lin_e_W, lin_e_b)
    return (h, e)


# speedup vs baseline: 1.3964x; 1.1916x over previous
import functools

import jax
import jax.numpy as jnp
from jax import lax
from jax.experimental import pallas as pl
from jax.experimental.pallas import tpu as pltpu
from jax.experimental.pallas import tpu_sc as plsc

_N_NODES = 50000
_N_EDGES = 800000
_NG = 50
_HID = 128
_EMB = 96
_TAGW = 32

_CTAB = 256


def _fuse_body(emb_ref, tagt_ref, w_ref, b_ref, c_ref):
    a = jnp.dot(emb_ref[:], w_ref[:_EMB, :], preferred_element_type=jnp.float32)
    bt = jnp.dot(tagt_ref[:], w_ref[_EMB:, :], preferred_element_type=jnp.float32)
    k = lax.broadcasted_iota(jnp.int32, (_CTAB, 1), 0)
    zi = k // 3
    ti = k - zi * 3
    oz = (lax.broadcasted_iota(jnp.int32, (_CTAB, 85), 1) == zi).astype(jnp.float32)
    ot = (lax.broadcasted_iota(jnp.int32, (_CTAB, 3), 1) == ti).astype(jnp.float32)
    c_ref[:] = (
        jnp.dot(oz, a, preferred_element_type=jnp.float32)
        + jnp.dot(ot, bt, preferred_element_type=jnp.float32)
        + b_ref[:]
    )


def _build_fused_table(emb_table, tag_table, lin_W, lin_b, interpret=False):
    return pl.pallas_call(
        _fuse_body,
        out_shape=jax.ShapeDtypeStruct((_CTAB, _HID), jnp.float32),
        interpret=interpret,
    )(emb_table, tag_table, lin_W, lin_b.reshape(1, _HID))


_BE = 32000


def _edge_body(rel_ref, attr_ref, w_ref, b_ref, out_ref):
    attr_t = attr_ref[:].astype(jnp.bfloat16)
    rel_t = rel_ref[:].astype(jnp.bfloat16)
    w = w_ref[:].astype(jnp.bfloat16)
    wr = w[0:3, :]
    wa = w[3:53, :]
    dn = (((0,), (0,)), ((), ()))
    out_ref[:] = (
        lax.dot_general(attr_t, wa, dn, preferred_element_type=jnp.float32)
        + lax.dot_general(rel_t, wr, dn, preferred_element_type=jnp.float32)
        + b_ref[:]
    )


def _edge_proj(rel_pos, edge_attr, lin_e_W, lin_e_b, interpret=False):
    n = rel_pos.shape[0]
    grid = (n // _BE,)
    return pl.pallas_call(
        _edge_body,
        grid=grid,
        in_specs=[
            pl.BlockSpec((3, _BE), lambda i: (0, i)),
            pl.BlockSpec((_NG, _BE), lambda i: (0, i)),
            pl.BlockSpec((_NG + 3, _HID), lambda i: (0, 0)),
            pl.BlockSpec((1, _HID), lambda i: (0, 0)),
        ],
        out_specs=pl.BlockSpec((_BE, _HID), lambda i: (i, 0)),
        out_shape=jax.ShapeDtypeStruct((n, _HID), jnp.float32),
        interpret=interpret,
    )(rel_pos.T, edge_attr.T, lin_e_W, lin_e_b.reshape(1, _HID))


_UNIT = 128
_UNITS_PER_W = 13
_FULL_W = 30
_ROWS_PER_W = _UNITS_PER_W * _UNIT
_TAIL_BASE = _FULL_W * _ROWS_PER_W
_TAIL = _N_NODES - _TAIL_BASE
_RING = 4
_INFLIGHT = 3


def _sc_gather_body(z_hbm, tag_hbm, c_hbm, out_hbm, z_v, tag_v, idx2_v, rows_v,
                    c_sh, gs0, gs1, gs2, gs3, ws0, ws1, ws2, ws3):
    c = lax.axis_index("c")
    s = lax.axis_index("s")
    w = s * 2 + c
    gsems = [gs0, gs1, gs2, gs3]
    wsems = [ws0, ws1, ws2, ws3]

    @pl.when(s == 0)
    def _():
        pltpu.sync_copy(c_hbm, c_sh)
    plsc.subcore_barrier()

    @pl.when(w < _FULL_W)
    def _():
        base0 = w * _ROWS_PER_W
        pltpu.sync_copy(z_hbm.at[pl.ds(base0, _ROWS_PER_W)], z_v)
        pltpu.sync_copy(tag_hbm.at[pl.ds(base0, _ROWS_PER_W)], tag_v)
        for u in range(_UNITS_PER_W):
            for j in range(_UNIT // 16):
                src = pl.ds(u * _UNIT + j * 16, 16)
                idx2_v[u, pl.ds(j * 16, 16)] = z_v[src] * 3 + tag_v[src]

        def gather(u):
            return pltpu.async_copy(
                c_sh.at[idx2_v.at[u]], rows_v.at[u % _RING], gsems[u % _RING])

        def writeback(u):
            return pltpu.async_copy(
                rows_v.at[u % _RING],
                out_hbm.at[pl.ds(base0 + u * _UNIT, _UNIT)],
                wsems[u % _RING])

        gs = {}
        wbs = {}
        for p in range(_INFLIGHT):
            gs[p] = gather(p)
        for u in range(_UNITS_PER_W):
            gs[u].wait()
            wbs[u] = writeback(u)
            nxt = u + _INFLIGHT
            if nxt < _UNITS_PER_W:
                if nxt >= _RING:
                    wbs[nxt - _RING].wait()
                gs[nxt] = gather(nxt)
        for u in range(_UNITS_PER_W - _RING, _UNITS_PER_W):
            wbs[u].wait()

    @pl.when(w == _FULL_W)
    def _():
        pltpu.sync_copy(z_hbm.at[pl.ds(_TAIL_BASE, _TAIL)], z_v.at[pl.ds(0, _TAIL)])
        pltpu.sync_copy(tag_hbm.at[pl.ds(_TAIL_BASE, _TAIL)], tag_v.at[pl.ds(0, _TAIL)])
        for j in range(_TAIL // 16):
            sl = pl.ds(j * 16, 16)
            idx2_v[0, sl] = z_v[sl] * 3 + tag_v[sl]
        pltpu.async_copy(
            c_sh.at[idx2_v.at[0, pl.ds(0, _TAIL)]],
            rows_v.at[0, pl.ds(0, _TAIL)], gs0).wait()
        pltpu.sync_copy(rows_v.at[0, pl.ds(0, _TAIL)],
                        out_hbm.at[pl.ds(_TAIL_BASE, _TAIL)])


def _node_gather(z, tag, c_table):
    mesh = plsc.VectorSubcoreMesh(core_axis_name="c", subcore_axis_name="s")
    return pl.kernel(
        _sc_gather_body,
        out_type=jax.ShapeDtypeStruct((_N_NODES, _HID), jnp.float32),
        mesh=mesh,
        scratch_types=[
            pltpu.VMEM((_ROWS_PER_W,), jnp.int32),
            pltpu.VMEM((_ROWS_PER_W,), jnp.int32),
            pltpu.VMEM((_UNITS_PER_W, _UNIT), jnp.int32),
            pltpu.VMEM((_RING, _UNIT, _HID), jnp.float32),
            pltpu.VMEM_SHARED((_CTAB, _HID), jnp.float32),
        ] + [pltpu.SemaphoreType.DMA] * 8,
    )(z, tag, c_table)


def kernel(z, rel_pos, edge_attr, tag, emb_table, tag_table, lin_W, lin_b, lin_e_W, lin_e_b):
    c_table = _build_fused_table(emb_table, tag_table, lin_W, lin_b)
    h = _node_gather(z.astype(jnp.int32), tag.astype(jnp.int32), c_table)
    e = _edge_proj(rel_pos, edge_attr, lin_e_W, lin_e_b)
    return (h, e)
